# Initial kernel scaffold; baseline (speedup 1.0000x reference)
import jax
import jax.numpy as jnp
from jax.experimental import pallas as pl


def kernel(x, index, weight, bias):
    raise NotImplementedError("write your pallas kernel here")


# speedup vs baseline: 11.1751x; 11.1751x over previous
import functools

import jax
import jax.numpy as jnp
from jax import lax
from jax.experimental import pallas as pl
from jax.experimental.pallas import tpu as pltpu
from jax.experimental.pallas import tpu_sc as plsc

NC = 2
NS = 16
NW = NC * NS
CH = 80
TB = 1280


def _mesh():
    return plsc.VectorSubcoreMesh(
        core_axis_name="c", subcore_axis_name="s", num_cores=NC,
        num_subcores=NS)


@functools.lru_cache(maxsize=None)
def _sc_degree(n_pad, e):
    ept = e // NW
    nch = ept // CH
    rows_pt = n_pad // NS

    def body(row_hbm, ones_hbm, zrow_hbm, degp_hbm, ones_v, idx_v, deg_sh):
        c = lax.axis_index("c")
        s = lax.axis_index("s")
        wid = s * NC + c
        pltpu.sync_copy(zrow_hbm, deg_sh.at[pl.ds(s * rows_pt, rows_pt)])
        pltpu.sync_copy(ones_hbm, ones_v)
        plsc.subcore_barrier()

        def step(i, carry):
            off = wid * ept + i * CH
            pltpu.sync_copy(row_hbm.at[pl.ds(off, CH)], idx_v)
            pltpu.sync_copy(ones_v, deg_sh.at[idx_v], add=True)
            return carry

        lax.fori_loop(0, nch, step, 0)
        plsc.subcore_barrier()
        pltpu.sync_copy(deg_sh.at[pl.ds(s * rows_pt, rows_pt)],
                        degp_hbm.at[pl.ds(c * n_pad + s * rows_pt, rows_pt)])

    return pl.kernel(
        body,
        out_type=jax.ShapeDtypeStruct((NC * n_pad,), jnp.float32),
        mesh=_mesh(),
        scratch_types=[
            pltpu.VMEM((CH,), jnp.float32),
            pltpu.VMEM((CH,), jnp.int32),
            pltpu.VMEM_SHARED((n_pad,), jnp.float32),
        ],
    )


@functools.lru_cache(maxsize=None)
def _sc_spmm(n_pad, f, e):
    ept = e // NW
    nch = ept // CH
    rows_pt = n_pad // NS
    zch = rows_pt // 5

    def body(tab_hbm, row_hbm, col_hbm, zblk_hbm, outp_hbm, cidx_v, ridx_v,
             rows_v, acc_sh, sem):
        c = lax.axis_index("c")
        s = lax.axis_index("s")
        wid = s * NC + c
        for j in range(5):
            pltpu.sync_copy(zblk_hbm,
                            acc_sh.at[pl.ds(s * rows_pt + j * zch, zch)])
        plsc.subcore_barrier()

        def step(i, carry):
            off = wid * ept + i * CH
            pltpu.sync_copy(col_hbm.at[pl.ds(off, CH)], cidx_v)
            gat = pltpu.async_copy(tab_hbm.at[cidx_v], rows_v, sem)
            pltpu.sync_copy(row_hbm.at[pl.ds(off, CH)], ridx_v)
            gat.wait()
            pltpu.sync_copy(rows_v, acc_sh.at[ridx_v], add=True)
            return carry

        lax.fori_loop(0, nch, step, 0)
        plsc.subcore_barrier()
        for j in range(5):
            src_off = s * rows_pt + j * zch
            pltpu.sync_copy(
                acc_sh.at[pl.ds(src_off, zch)],
                outp_hbm.at[pl.ds(c * n_pad + src_off, zch)])

    return pl.kernel(
        body,
        out_type=jax.ShapeDtypeStruct((NC * n_pad, f), jnp.float32),
        mesh=_mesh(),
        scratch_types=[
            pltpu.VMEM((CH,), jnp.int32),
            pltpu.VMEM((CH,), jnp.int32),
            pltpu.VMEM((CH, f), jnp.float32),
            pltpu.VMEM_SHARED((n_pad, f), jnp.float32),
            pltpu.SemaphoreType.DMA,
        ],
    )


def _dis(degpt_ref):
    d = degpt_ref[:, 0:1] + degpt_ref[:, 1:2]
    return jnp.where(d > 0, lax.rsqrt(d), 0.0)


def _prep_body(x_ref, degpt_ref, u_ref):
    u_ref[...] = x_ref[...] * _dis(degpt_ref)


def _mid_body(vp0_ref, vp1_ref, degpt_ref, t1_ref, w_ref):
    dis = _dis(degpt_ref)
    t1 = -((vp0_ref[...] + vp1_ref[...]) * dis)
    t1_ref[...] = t1
    w_ref[...] = t1 * dis


def _final_body(x_ref, t1_ref, zp0_ref, zp1_ref, degpt_ref, wt_ref, b_ref,
                o_ref):
    dis = _dis(degpt_ref)
    srow = -((zp0_ref[...] + zp1_ref[...]) * dis)
    hi = jax.lax.Precision.HIGHEST
    acc = jnp.dot(x_ref[...], wt_ref[0] - wt_ref[2],
                  preferred_element_type=jnp.float32, precision=hi)
    acc = acc + jnp.dot(t1_ref[...], wt_ref[1],
                        preferred_element_type=jnp.float32, precision=hi)
    acc = acc + jnp.dot(srow, 2.0 * wt_ref[2],
                        preferred_element_type=jnp.float32, precision=hi)
    o_ref[...] = acc + b_ref[...]


def _row_blk(f):
    return pl.BlockSpec((TB, f), lambda i: (i, 0))


def _row_blk_hi(n_pad, f):
    return pl.BlockSpec((TB, f), lambda i: (i + n_pad // TB, 0))


def kernel(x, index, weight, bias):
    n, f = x.shape
    e = index.shape[1]
    blk = NS * 8 * 5 * 2
    n_pad = ((n + blk - 1) // blk) * blk
    row = index[0]
    col = index[1]
    xp = jnp.pad(x, ((0, n_pad - n), (0, 0)))
    grid = (n_pad // TB,)

    ones_row = jnp.ones((CH,), jnp.float32)
    zero_row = jnp.zeros((n_pad // NS,), jnp.float32)
    zero_blk = jnp.zeros((n_pad // NS // 5, f), jnp.float32)

    degp = _sc_degree(n_pad, e)(row, ones_row, zero_row)
    degpt = jnp.stack([degp[:n_pad], degp[n_pad:]], axis=1)

    u = pl.pallas_call(
        _prep_body,
        grid=grid,
        in_specs=[_row_blk(f), _row_blk(2)],
        out_specs=_row_blk(f),
        out_shape=jax.ShapeDtypeStruct((n_pad, f), jnp.float32),
    )(xp, degpt)

    vp = _sc_spmm(n_pad, f, e)(u, row, col, zero_blk)

    t1, w = pl.pallas_call(
        _mid_body,
        grid=grid,
        in_specs=[_row_blk(f), _row_blk_hi(n_pad, f), _row_blk(2)],
        out_specs=(_row_blk(f), _row_blk(f)),
        out_shape=(jax.ShapeDtypeStruct((n_pad, f), jnp.float32),
                   jax.ShapeDtypeStruct((n_pad, f), jnp.float32)),
    )(vp, vp, degpt)

    zp = _sc_spmm(n_pad, f, e)(w, row, col, zero_blk)

    out = pl.pallas_call(
        _final_body,
        grid=grid,
        in_specs=[_row_blk(f), _row_blk(f), _row_blk(f),
                  _row_blk_hi(n_pad, f), _row_blk(2),
                  pl.BlockSpec((3, f, f), lambda i: (0, 0, 0)),
                  pl.BlockSpec((1, f), lambda i: (0, 0))],
        out_specs=_row_blk(f),
        out_shape=jax.ShapeDtypeStruct((n_pad, f), jnp.float32),
    )(xp, t1, zp, zp, degpt, weight, bias.reshape(1, f))
    return out[:n]
